# Initial kernel scaffold; baseline (speedup 1.0000x reference)
import jax
import jax.numpy as jnp
from jax.experimental import pallas as pl
from jax.experimental.pallas import tpu as pltpu

N = 100000
D = 128
K = 64
BLK = 1024
GRID = (N + BLK - 1) // BLK

NEG = jnp.float32(-jnp.inf)


def _ce_lane(x, d, take_min, low):
    a = pltpu.roll(x, -d, 1)
    b = pltpu.roll(x, d, 1)
    xp = jnp.where(low, a, b)
    return jnp.where(take_min, jnp.minimum(x, xp), jnp.maximum(x, xp))


def _ce_sub(x, d, take_max, low):
    a = pltpu.roll(x, -d, 0)
    b = pltpu.roll(x, d, 0)
    xp = jnp.where(low, a, b)
    return jnp.where(take_max, jnp.maximum(x, xp), jnp.minimum(x, xp))


def _sort_rows_asc(x, lane):
    kk = 2
    while kk <= D:
        desc = (lane & kk) != 0
        d = kk // 2
        while d >= 1:
            low = (lane & d) == 0
            take_min = jnp.logical_xor(low, desc)
            x = _ce_lane(x, d, take_min, low)
            d //= 2
        kk *= 2
    return x


def _sort64_desc(x, row):
    kk = 2
    while kk <= K:
        asc = (row & kk) != 0
        d = kk // 2
        while d >= 1:
            low = (row & d) == 0
            take_max = jnp.logical_xor(low, asc)
            x = _ce_sub(x, d, take_max, low)
            d //= 2
        kk *= 2
    return x


def _merge64_desc(a, b, row):
    c = jnp.maximum(a, jnp.flip(b, 0))
    d = K // 2
    while d >= 1:
        low = (row & d) == 0
        c = _ce_sub(c, d, low, low)
        d //= 2
    return c


def _tc_body(x_ref, o_ref, acc_ref):
    i = pl.program_id(0)
    x = x_ref[...]
    rowg = jax.lax.broadcasted_iota(jnp.int32, (BLK, 1), 0) + i * BLK
    x = jnp.where(rowg < N, x, NEG)
    lane = jax.lax.broadcasted_iota(jnp.int32, (1, D), 1)
    x = _sort_rows_asc(x, lane)

    row = jax.lax.broadcasted_iota(jnp.int32, (K, 1), 0)
    tiles = [_sort64_desc(x[t * K:(t + 1) * K, :], row) for t in range(BLK // K)]
    while len(tiles) > 1:
        tiles = [_merge64_desc(tiles[2 * j], tiles[2 * j + 1], row)
                 for j in range(len(tiles) // 2)]
    top = tiles[0]

    @pl.when(i == 0)
    def _():
        acc_ref[...] = top

    @pl.when(i > 0)
    def _():
        acc_ref[...] = _merge64_desc(acc_ref[...], top, row)

    @pl.when(i == GRID - 1)
    def _():
        o_ref[...] = acc_ref[...]


def _run_tc(feat, interpret=False):
    return pl.pallas_call(
        _tc_body,
        grid=(GRID,),
        in_specs=[pl.BlockSpec((BLK, D), lambda i: (i, 0))],
        out_specs=pl.BlockSpec((K, D), lambda i: (0, 0)),
        out_shape=jax.ShapeDtypeStruct((K, D), jnp.float32),
        scratch_shapes=[pltpu.VMEM((K, D), jnp.float32)],
        compiler_params=pltpu.CompilerParams(
            dimension_semantics=("arbitrary",)),
        interpret=interpret,
    )(feat)


@jax.jit
def kernel(feat):
    return _run_tc(feat).reshape(K * D)


# speedup vs baseline: 3.2645x; 3.2645x over previous
import jax
import jax.numpy as jnp
from jax.experimental import pallas as pl
from jax.experimental.pallas import tpu as pltpu

N = 100000
D = 128
K = 64
BLK = 1024
GRID = (N + BLK - 1) // BLK

NEG = float("-inf")


def _ce_lane(x, d, take_min, low):
    a = pltpu.roll(x, D - d, 1)
    b = pltpu.roll(x, d, 1)
    xp = jnp.where(low, a, b)
    return jnp.where(take_min, jnp.minimum(x, xp), jnp.maximum(x, xp))


def _ce_sub(x, d, take_max, low):
    a = pltpu.roll(x, K - d, 0)
    b = pltpu.roll(x, d, 0)
    xp = jnp.where(low, a, b)
    return jnp.where(take_max, jnp.maximum(x, xp), jnp.minimum(x, xp))


def _sort_rows_asc(x, lane):
    kk = 2
    while kk <= D:
        desc = (lane & kk) != 0
        d = kk // 2
        while d >= 1:
            low = (lane & d) == 0
            take_min = jnp.logical_xor(low, desc)
            x = _ce_lane(x, d, take_min, low)
            d //= 2
        kk *= 2
    return x


def _sort64(x, row, desc):
    kk = 2
    while kk <= K:
        blk = (row & kk) != 0
        d = kk // 2
        while d >= 1:
            low = (row & d) == 0
            m = jnp.logical_xor(low, blk)
            take_max = m if desc else jnp.logical_not(m)
            x = _ce_sub(x, d, take_max, low)
            d //= 2
        kk *= 2
    return x


def _clean64(c, row, desc):
    d = K // 2
    while d >= 1:
        low = (row & d) == 0
        take_max = low if desc else jnp.logical_not(low)
        c = _ce_sub(c, d, take_max, low)
        d //= 2
    return c


def _merge64(a_desc, b_asc, row, desc):
    return _clean64(jnp.maximum(a_desc, b_asc), row, desc)


def _block_top64(tiles, row, desc):
    if len(tiles) == 1:
        return _sort64(tiles[0], row, desc)
    h = len(tiles) // 2
    a = _block_top64(tiles[:h], row, True)
    b = _block_top64(tiles[h:], row, False)
    return _merge64(a, b, row, desc)


def _tc_body(x_ref, o_ref, acc_ref):
    i = pl.program_id(0)
    x = x_ref[...]
    rowg = jax.lax.broadcasted_iota(jnp.int32, (BLK, 1), 0) + i * BLK
    x = jnp.where(rowg < N, x, NEG)
    lane = jax.lax.broadcasted_iota(jnp.int32, (1, D), 1)
    x = _sort_rows_asc(x, lane)

    row = jax.lax.broadcasted_iota(jnp.int32, (K, 1), 0)
    tiles = [x[t * K:(t + 1) * K, :] for t in range(BLK // K)]
    top = _block_top64(tiles, row, desc=False)

    prev = jnp.where(i == 0, NEG, acc_ref[...])
    acc_ref[...] = _merge64(prev, top, row, desc=True)

    @pl.when(i == GRID - 1)
    def _():
        o_ref[...] = acc_ref[...]


def _run_tc(feat, interpret=False):
    return pl.pallas_call(
        _tc_body,
        grid=(GRID,),
        in_specs=[pl.BlockSpec((BLK, D), lambda i: (i, 0))],
        out_specs=pl.BlockSpec((K, D), lambda i: (0, 0)),
        out_shape=jax.ShapeDtypeStruct((K, D), jnp.float32),
        scratch_shapes=[pltpu.VMEM((K, D), jnp.float32)],
        compiler_params=pltpu.CompilerParams(
            dimension_semantics=("arbitrary",)),
        interpret=interpret,
    )(feat)


@jax.jit
def kernel(feat):
    return _run_tc(feat).reshape(K * D)
